# baseline (speedup 1.0000x reference)
import functools

import jax
import jax.numpy as jnp
from jax import lax
from jax.experimental import pallas as pl
from jax.experimental.pallas import tpu as pltpu
from jax.experimental.pallas import tpu_sc as plsc

NUM_ROLES = 100
NUM_TOKENS = 50
EMBED_DIM = 4096

NC = 2
NS = 16
NW = NC * NS
SUB = 1
NSUB = NW * SUB
MINOR = NUM_TOKENS * EMBED_DIM // NSUB

_mesh = plsc.VectorSubcoreMesh(core_axis_name="c", subcore_axis_name="s")


@functools.partial(
    pl.kernel,
    mesh=_mesh,
    out_type=jax.ShapeDtypeStruct((NSUB, MINOR), jnp.float32),
    scratch_types=[
        pltpu.VMEM((SUB,), jnp.int32),
        pltpu.VMEM((SUB, MINOR), jnp.float32),
        pltpu.SemaphoreType.DMA,
    ],
)
def _sc_select(table_hbm, idx_hbm, out_hbm, idx_v, rows_v, sem):
    wid = lax.axis_index("s") * NC + lax.axis_index("c")
    pltpu.sync_copy(idx_hbm.at[wid], idx_v)
    pltpu.async_copy(table_hbm.at[idx_v], rows_v, sem).wait()
    pltpu.sync_copy(rows_v, out_hbm.at[pl.ds(wid * SUB, SUB)])


def kernel(embeds, role_id):
    table = embeds.reshape(NUM_ROLES * NSUB, MINOR)
    rid = jnp.asarray(role_id, jnp.int32)
    idx = (rid * NSUB + lax.iota(jnp.int32, NSUB)).reshape(NW, SUB)
    out = _sc_select(table, idx)
    return out.reshape(NUM_TOKENS, EMBED_DIM)


# speedup vs baseline: 1.3484x; 1.3484x over previous
import functools

import jax
import jax.numpy as jnp
from jax import lax
from jax.experimental import pallas as pl
from jax.experimental.pallas import tpu as pltpu
from jax.experimental.pallas import tpu_sc as plsc

NUM_ROLES = 100
NUM_TOKENS = 50
EMBED_DIM = 4096

NC = 2
NS = 16
NW = NC * NS
CH = EMBED_DIM // NW

_mesh = plsc.VectorSubcoreMesh(core_axis_name="c", subcore_axis_name="s")


@functools.partial(
    pl.kernel,
    mesh=_mesh,
    out_type=jax.ShapeDtypeStruct((NUM_TOKENS, EMBED_DIM), jnp.float32),
    scratch_types=[
        pltpu.VMEM((16,), jnp.int32),
    ],
)
def _sc_select(table_hbm, rid_hbm, out_hbm, rid_v):
    wid = lax.axis_index("s") * NC + lax.axis_index("c")
    pltpu.sync_copy(rid_hbm, rid_v)
    rid = rid_v[...][0]
    col = wid * CH
    pltpu.sync_copy(
        table_hbm.at[rid, :, pl.ds(col, CH)],
        out_hbm.at[:, pl.ds(col, CH)],
    )


def kernel(embeds, role_id):
    rid16 = jnp.full((16,), role_id, dtype=jnp.int32)
    return _sc_select(embeds, rid16)
